# Initial kernel scaffold; baseline (speedup 1.0000x reference)
import jax
import jax.numpy as jnp
from jax.experimental import pallas as pl


def kernel(x, edge_index, edge_attr, batch, Wq, bq, Wk, bk, Wv, bv, We, be, Wskip, bskip, norm_w, norm_b, norm_ms, film_s, film_t, phi_W1, phi_b1, phi_W2, phi_b2, Wout, bout, bias):
    raise NotImplementedError("write your pallas kernel here")


# speedup vs baseline: 1.6624x; 1.6624x over previous
import jax
import jax.numpy as jnp
from jax import lax
from jax.experimental import pallas as pl

TN = 2000
TE = 2000
NB = 64


def _pair_mat():
    r = lax.broadcasted_iota(jnp.int32, (8, 4), 0)
    c = lax.broadcasted_iota(jnp.int32, (8, 4), 1)
    return (r // 2 == c).astype(jnp.float32)


def _onehot(b):
    i = lax.broadcasted_iota(jnp.int32, (b.shape[0], NB), 1)
    return (b == i).astype(jnp.float32)


def _proj_body(h_ref, wq_ref, bq_ref, wk_ref, bk_ref, wv_ref, bv_ref,
               ws_ref, bs_ref, q_ref, k_ref, v_ref, s_ref):
    h = h_ref[...]

    def mm(w_ref, b_ref):
        return lax.dot_general(h, w_ref[...], (((1,), (1,)), ((), ())),
                               preferred_element_type=jnp.float32, precision=lax.Precision.HIGHEST) + b_ref[...]

    q_ref[...] = mm(wq_ref, bq_ref)
    k_ref[...] = mm(wk_ref, bk_ref)
    v_ref[...] = mm(wv_ref, bv_ref)
    s_ref[...] = mm(ws_ref, bs_ref)


def _edge1_body(qd_ref, ks_ref, ea_ref, we_ref, be_ref, alpha_ref, ep_ref):
    ep = lax.dot_general(ea_ref[...], we_ref[...], (((1,), (1,)), ((), ())),
                         preferred_element_type=jnp.float32, precision=lax.Precision.HIGHEST) + be_ref[...]
    ep_ref[...] = ep
    mul = qd_ref[...] * (ks_ref[...] + ep)
    P = _pair_mat()
    alpha_ref[...] = lax.dot_general(
        mul, P, (((1,), (0,)), ((), ())),
        preferred_element_type=jnp.float32, precision=lax.Precision.HIGHEST) * (1.0 / jnp.sqrt(2.0))


def _edge2_body(a_ref, md_ref, ex_ref):
    ex_ref[...] = jnp.exp(a_ref[...] - md_ref[...])


def _edge3_body(ex_ref, den_ref, vs_ref, ep_ref, msg_ref):
    a = ex_ref[...] / (den_ref[...] + 1e-16)
    P = _pair_mat()
    a8 = lax.dot_general(a, P, (((1,), (1,)), ((), ())),
                         preferred_element_type=jnp.float32, precision=lax.Precision.HIGHEST)
    msg_ref[...] = (vs_ref[...] + ep_ref[...]) * a8


def _seg_body(b_ref, v_ref, af_ref, ac_ref):
    oh = _onehot(b_ref[...])
    @pl.when(pl.program_id(0) == 0)
    def _():
        af_ref[...] = jnp.zeros_like(af_ref)
        ac_ref[...] = jnp.zeros_like(ac_ref)
    af_ref[...] += lax.dot_general(oh, v_ref[...], (((0,), (0,)), ((), ())),
                                   preferred_element_type=jnp.float32, precision=lax.Precision.HIGHEST)
    ones = jnp.ones((v_ref.shape[0], 1), jnp.float32)
    ac_ref[...] += lax.dot_general(oh, ones, (((0,), (0,)), ((), ())),
                                   preferred_element_type=jnp.float32, precision=lax.Precision.HIGHEST)


def _bn1_body(agg_ref, sk_ref, b_ref, out_ref, sum_ref):
    out = agg_ref[...] + sk_ref[...]
    out_ref[...] = out
    oh = _onehot(b_ref[...])
    @pl.when(pl.program_id(0) == 0)
    def _():
        sum_ref[...] = jnp.zeros_like(sum_ref)
    sum_ref[...] += lax.dot_general(oh, out, (((0,), (0,)), ((), ())),
                                    preferred_element_type=jnp.float32, precision=lax.Precision.HIGHEST)


def _bn2_body(out_ref, b_ref, mean_ref, nms_ref, cen_ref, c2_ref):
    oh = _onehot(b_ref[...])
    mu = lax.dot_general(oh, mean_ref[...], (((1,), (0,)), ((), ())),
                         preferred_element_type=jnp.float32, precision=lax.Precision.HIGHEST)
    cen = out_ref[...] - mu * nms_ref[...]
    cen_ref[...] = cen
    @pl.when(pl.program_id(0) == 0)
    def _():
        c2_ref[...] = jnp.zeros_like(c2_ref)
    c2_ref[...] += lax.dot_general(oh, cen * cen, (((0,), (0,)), ((), ())),
                                   preferred_element_type=jnp.float32, precision=lax.Precision.HIGHEST)


def _bn3_body(cen_ref, b_ref, var_ref, gf_ref, nw_ref, nb_ref, fs_ref, ft_ref,
              h_ref):
    oh = _onehot(b_ref[...])
    vr = lax.dot_general(oh, var_ref[...], (((1,), (0,)), ((), ())),
                         preferred_element_type=jnp.float32, precision=lax.Precision.HIGHEST)
    z = cen_ref[...] / jnp.sqrt(vr + 1e-5)
    z = z * nw_ref[...] + nb_ref[...]
    gn = lax.dot_general(oh, gf_ref[...], (((1,), (0,)), ((), ())),
                         preferred_element_type=jnp.float32, precision=lax.Precision.HIGHEST)
    z = z * (1.0 + gn * fs_ref[...]) + gn * ft_ref[...]
    h_ref[...] = jnp.where(z > 0.0, z, jnp.exp(jnp.minimum(z, 0.0)) - 1.0)


def _read_body(h_ref, b_ref, acc_ref):
    h = h_ref[...]
    m = jnp.max(h, axis=1, keepdims=True)
    e = jnp.exp(h - m)
    s = jnp.sum(e, axis=1, keepdims=True)
    contrib = (e / s) * h
    oh = _onehot(b_ref[...])
    @pl.when(pl.program_id(0) == 0)
    def _():
        acc_ref[...] = jnp.zeros_like(acc_ref)
    acc_ref[...] += lax.dot_general(oh, contrib, (((0,), (0,)), ((), ())),
                                    preferred_element_type=jnp.float32, precision=lax.Precision.HIGHEST)


def _tile(T, w):
    return pl.BlockSpec((T, w), lambda i: (i, 0))


def _full(shape):
    return pl.BlockSpec(shape, lambda i: (0, 0))


def kernel(x, edge_index, edge_attr, batch, Wq, bq, Wk, bk, Wv, bv, We, be,
           Wskip, bskip, norm_w, norm_b, norm_ms, film_s, film_t, phi_W1,
           phi_b1, phi_W2, phi_b2, Wout, bout, bias):
    N, D = x.shape
    E = edge_index.shape[1]
    L = Wq.shape[0]
    f32 = jnp.float32
    src = edge_index[0]
    dst = edge_index[1]
    gn_steps = N // TN
    ge_steps = E // TE

    batch2 = batch.reshape(N, 1)
    eb2 = batch[src].reshape(E, 1)
    ef2 = edge_attr[:, 2:3]
    ea8 = jnp.pad(edge_attr, ((0, 0), (0, 8 - edge_attr.shape[1])))
    We8 = jnp.pad(We, ((0, 0), (0, 0), (0, 8 - We.shape[2])))

    seg_edges = pl.pallas_call(
        _seg_body, grid=(ge_steps,),
        in_specs=[_tile(TE, 1), _tile(TE, 1)],
        out_shape=[jax.ShapeDtypeStruct((NB, 1), f32),
                   jax.ShapeDtypeStruct((NB, 1), f32)],
        out_specs=[_full((NB, 1)), _full((NB, 1))])
    fsum, ecnt = seg_edges(eb2, ef2)
    seg_nodes = pl.pallas_call(
        _seg_body, grid=(gn_steps,),
        in_specs=[_tile(TN, 1), _tile(TN, 1)],
        out_shape=[jax.ShapeDtypeStruct((NB, 1), f32),
                   jax.ShapeDtypeStruct((NB, 1), f32)],
        out_specs=[_full((NB, 1)), _full((NB, 1))])
    _, ncnt2 = seg_nodes(batch2, batch2.astype(f32))

    flying_range = fsum[:, 0] / jnp.maximum(ecnt[:, 0], 1.0)
    u = jnp.maximum(flying_range[:, None] @ phi_W1.T + phi_b1, 0.0)
    u = u @ phi_W2.T + phi_b2
    phi = jax.nn.softplus(u[:, 0])
    gf2 = (flying_range * phi).reshape(NB, 1)
    ncnt = jnp.maximum(ncnt2, 1.0)

    proj = pl.pallas_call(
        _proj_body, grid=(gn_steps,),
        in_specs=[_tile(TN, D)] + [_full((D, D)), _full((1, D))] * 4,
        out_shape=[jax.ShapeDtypeStruct((N, D), f32)] * 4,
        out_specs=[_tile(TN, D)] * 4)
    edge1 = pl.pallas_call(
        _edge1_body, grid=(ge_steps,),
        in_specs=[_tile(TE, D), _tile(TE, D), _tile(TE, D),
                  _full((D, D)), _full((1, D))],
        out_shape=[jax.ShapeDtypeStruct((E, 4), f32),
                   jax.ShapeDtypeStruct((E, D), f32)],
        out_specs=[_tile(TE, 4), _tile(TE, D)])
    edge2 = pl.pallas_call(
        _edge2_body, grid=(ge_steps,),
        in_specs=[_tile(TE, 4), _tile(TE, 4)],
        out_shape=jax.ShapeDtypeStruct((E, 4), f32),
        out_specs=_tile(TE, 4))
    edge3 = pl.pallas_call(
        _edge3_body, grid=(ge_steps,),
        in_specs=[_tile(TE, 4), _tile(TE, 4), _tile(TE, D), _tile(TE, D)],
        out_shape=jax.ShapeDtypeStruct((E, D), f32),
        out_specs=_tile(TE, D))
    bn1 = pl.pallas_call(
        _bn1_body, grid=(gn_steps,),
        in_specs=[_tile(TN, D), _tile(TN, D), _tile(TN, 1)],
        out_shape=[jax.ShapeDtypeStruct((N, D), f32),
                   jax.ShapeDtypeStruct((NB, D), f32)],
        out_specs=[_tile(TN, D), _full((NB, D))])
    bn2 = pl.pallas_call(
        _bn2_body, grid=(gn_steps,),
        in_specs=[_tile(TN, D), _tile(TN, 1), _full((NB, D)), _full((1, D))],
        out_shape=[jax.ShapeDtypeStruct((N, D), f32),
                   jax.ShapeDtypeStruct((NB, D), f32)],
        out_specs=[_tile(TN, D), _full((NB, D))])
    bn3 = pl.pallas_call(
        _bn3_body, grid=(gn_steps,),
        in_specs=[_tile(TN, D), _tile(TN, 1), _full((NB, D)), _full((NB, 1))]
        + [_full((1, D))] * 4,
        out_shape=jax.ShapeDtypeStruct((N, D), f32),
        out_specs=_tile(TN, D))
    readout = pl.pallas_call(
        _read_body, grid=(gn_steps,),
        in_specs=[_tile(TN, D), _tile(TN, 1)],
        out_shape=jax.ShapeDtypeStruct((NB, D), f32),
        out_specs=_full((NB, D)))

    h = x
    for l in range(L):
        q, k, v, sk = proj(h, Wq[l], bq[l].reshape(1, D), Wk[l],
                           bk[l].reshape(1, D), Wv[l], bv[l].reshape(1, D),
                           Wskip[l], bskip[l].reshape(1, D))
        alpha, ep = edge1(q[dst], k[src], ea8, We8[l], be[l].reshape(1, D))
        m = jax.ops.segment_max(alpha, dst, num_segments=N)
        m = jnp.where(jnp.isfinite(m), m, 0.0)
        ex = edge2(alpha, m[dst])
        den = jax.ops.segment_sum(ex, dst, num_segments=N)
        msg = edge3(ex, den[dst], v[src], ep)
        agg = jax.ops.segment_sum(msg, dst, num_segments=N)
        out, osum = bn1(agg, sk, batch2)
        mean = osum / ncnt
        cen, c2 = bn2(out, batch2, mean, norm_ms[l].reshape(1, D))
        var = c2 / ncnt
        h = bn3(cen, batch2, var, gf2, norm_w[l].reshape(1, D),
                norm_b[l].reshape(1, D), film_s[l].reshape(1, D),
                film_t[l].reshape(1, D))

    hG = readout(h, batch2)
    return hG @ Wout.T + bout + bias


# speedup vs baseline: 1.6776x; 1.0091x over previous
import jax
import jax.numpy as jnp
from jax import lax
from jax.experimental import pallas as pl

TN = 5000
TE = 4000
NB = 64


def _pair_mat():
    r = lax.broadcasted_iota(jnp.int32, (8, 4), 0)
    c = lax.broadcasted_iota(jnp.int32, (8, 4), 1)
    return (r // 2 == c).astype(jnp.float32)


def _onehot(b):
    i = lax.broadcasted_iota(jnp.int32, (b.shape[0], NB), 1)
    return (b == i).astype(jnp.float32)


def _proj_body(h_ref, wq_ref, bq_ref, wk_ref, bk_ref, wv_ref, bv_ref,
               ws_ref, bs_ref, q_ref, k_ref, v_ref, s_ref):
    h = h_ref[...]

    def mm(w_ref, b_ref):
        return lax.dot_general(h, w_ref[...], (((1,), (1,)), ((), ())),
                               preferred_element_type=jnp.float32, precision=lax.Precision.HIGHEST) + b_ref[...]

    q_ref[...] = mm(wq_ref, bq_ref)
    k_ref[...] = mm(wk_ref, bk_ref)
    v_ref[...] = mm(wv_ref, bv_ref)
    s_ref[...] = mm(ws_ref, bs_ref)


def _edge1_body(qd_ref, ks_ref, ea_ref, we_ref, be_ref, alpha_ref, ep_ref):
    ep = lax.dot_general(ea_ref[...], we_ref[...], (((1,), (1,)), ((), ())),
                         preferred_element_type=jnp.float32, precision=lax.Precision.HIGHEST) + be_ref[...]
    ep_ref[...] = ep
    mul = qd_ref[...] * (ks_ref[...] + ep)
    P = _pair_mat()
    alpha_ref[...] = lax.dot_general(
        mul, P, (((1,), (0,)), ((), ())),
        preferred_element_type=jnp.float32, precision=lax.Precision.HIGHEST) * (1.0 / jnp.sqrt(2.0))


def _edge2_body(a_ref, md_ref, ex_ref):
    ex_ref[...] = jnp.exp(a_ref[...] - md_ref[...])


def _edge3_body(ex_ref, den_ref, vs_ref, ep_ref, msg_ref):
    a = ex_ref[...] / (den_ref[...] + 1e-16)
    P = _pair_mat()
    a8 = lax.dot_general(a, P, (((1,), (1,)), ((), ())),
                         preferred_element_type=jnp.float32, precision=lax.Precision.HIGHEST)
    msg_ref[...] = (vs_ref[...] + ep_ref[...]) * a8


def _seg_body(b_ref, v_ref, af_ref, ac_ref):
    oh = _onehot(b_ref[...])
    @pl.when(pl.program_id(0) == 0)
    def _():
        af_ref[...] = jnp.zeros_like(af_ref)
        ac_ref[...] = jnp.zeros_like(ac_ref)
    af_ref[...] += lax.dot_general(oh, v_ref[...], (((0,), (0,)), ((), ())),
                                   preferred_element_type=jnp.float32, precision=lax.Precision.HIGHEST)
    ones = jnp.ones((v_ref.shape[0], 1), jnp.float32)
    ac_ref[...] += lax.dot_general(oh, ones, (((0,), (0,)), ((), ())),
                                   preferred_element_type=jnp.float32, precision=lax.Precision.HIGHEST)


def _bn1_body(agg_ref, sk_ref, b_ref, out_ref, sum_ref):
    out = agg_ref[...] + sk_ref[...]
    out_ref[...] = out
    oh = _onehot(b_ref[...])
    @pl.when(pl.program_id(0) == 0)
    def _():
        sum_ref[...] = jnp.zeros_like(sum_ref)
    sum_ref[...] += lax.dot_general(oh, out, (((0,), (0,)), ((), ())),
                                    preferred_element_type=jnp.float32, precision=lax.Precision.HIGHEST)


def _bn2_body(out_ref, b_ref, mean_ref, nms_ref, cen_ref, c2_ref):
    oh = _onehot(b_ref[...])
    mu = lax.dot_general(oh, mean_ref[...], (((1,), (0,)), ((), ())),
                         preferred_element_type=jnp.float32, precision=lax.Precision.HIGHEST)
    cen = out_ref[...] - mu * nms_ref[...]
    cen_ref[...] = cen
    @pl.when(pl.program_id(0) == 0)
    def _():
        c2_ref[...] = jnp.zeros_like(c2_ref)
    c2_ref[...] += lax.dot_general(oh, cen * cen, (((0,), (0,)), ((), ())),
                                   preferred_element_type=jnp.float32, precision=lax.Precision.HIGHEST)


def _bn3_body(cen_ref, b_ref, var_ref, gf_ref, nw_ref, nb_ref, fs_ref, ft_ref,
              h_ref):
    oh = _onehot(b_ref[...])
    vr = lax.dot_general(oh, var_ref[...], (((1,), (0,)), ((), ())),
                         preferred_element_type=jnp.float32, precision=lax.Precision.HIGHEST)
    z = cen_ref[...] / jnp.sqrt(vr + 1e-5)
    z = z * nw_ref[...] + nb_ref[...]
    gn = lax.dot_general(oh, gf_ref[...], (((1,), (0,)), ((), ())),
                         preferred_element_type=jnp.float32, precision=lax.Precision.HIGHEST)
    z = z * (1.0 + gn * fs_ref[...]) + gn * ft_ref[...]
    h_ref[...] = jnp.where(z > 0.0, z, jnp.exp(jnp.minimum(z, 0.0)) - 1.0)


def _read_body(h_ref, b_ref, acc_ref):
    h = h_ref[...]
    m = jnp.max(h, axis=1, keepdims=True)
    e = jnp.exp(h - m)
    s = jnp.sum(e, axis=1, keepdims=True)
    contrib = (e / s) * h
    oh = _onehot(b_ref[...])
    @pl.when(pl.program_id(0) == 0)
    def _():
        acc_ref[...] = jnp.zeros_like(acc_ref)
    acc_ref[...] += lax.dot_general(oh, contrib, (((0,), (0,)), ((), ())),
                                    preferred_element_type=jnp.float32, precision=lax.Precision.HIGHEST)


def _tile(T, w):
    return pl.BlockSpec((T, w), lambda i: (i, 0))


def _full(shape):
    return pl.BlockSpec(shape, lambda i: (0, 0))


def kernel(x, edge_index, edge_attr, batch, Wq, bq, Wk, bk, Wv, bv, We, be,
           Wskip, bskip, norm_w, norm_b, norm_ms, film_s, film_t, phi_W1,
           phi_b1, phi_W2, phi_b2, Wout, bout, bias):
    N, D = x.shape
    E = edge_index.shape[1]
    L = Wq.shape[0]
    f32 = jnp.float32
    src = edge_index[0]
    dst = edge_index[1]
    gn_steps = N // TN
    ge_steps = E // TE

    batch2 = batch.reshape(N, 1)
    eb2 = batch[src].reshape(E, 1)
    ef2 = edge_attr[:, 2:3]
    ea8 = jnp.pad(edge_attr, ((0, 0), (0, 8 - edge_attr.shape[1])))
    We8 = jnp.pad(We, ((0, 0), (0, 0), (0, 8 - We.shape[2])))

    seg_edges = pl.pallas_call(
        _seg_body, grid=(ge_steps,),
        in_specs=[_tile(TE, 1), _tile(TE, 1)],
        out_shape=[jax.ShapeDtypeStruct((NB, 1), f32),
                   jax.ShapeDtypeStruct((NB, 1), f32)],
        out_specs=[_full((NB, 1)), _full((NB, 1))])
    fsum, ecnt = seg_edges(eb2, ef2)
    seg_nodes = pl.pallas_call(
        _seg_body, grid=(gn_steps,),
        in_specs=[_tile(TN, 1), _tile(TN, 1)],
        out_shape=[jax.ShapeDtypeStruct((NB, 1), f32),
                   jax.ShapeDtypeStruct((NB, 1), f32)],
        out_specs=[_full((NB, 1)), _full((NB, 1))])
    _, ncnt2 = seg_nodes(batch2, batch2.astype(f32))

    flying_range = fsum[:, 0] / jnp.maximum(ecnt[:, 0], 1.0)
    u = jnp.maximum(flying_range[:, None] @ phi_W1.T + phi_b1, 0.0)
    u = u @ phi_W2.T + phi_b2
    phi = jax.nn.softplus(u[:, 0])
    gf2 = (flying_range * phi).reshape(NB, 1)
    ncnt = jnp.maximum(ncnt2, 1.0)

    proj = pl.pallas_call(
        _proj_body, grid=(gn_steps,),
        in_specs=[_tile(TN, D)] + [_full((D, D)), _full((1, D))] * 4,
        out_shape=[jax.ShapeDtypeStruct((N, D), f32)] * 4,
        out_specs=[_tile(TN, D)] * 4)
    edge1 = pl.pallas_call(
        _edge1_body, grid=(ge_steps,),
        in_specs=[_tile(TE, D), _tile(TE, D), _tile(TE, D),
                  _full((D, D)), _full((1, D))],
        out_shape=[jax.ShapeDtypeStruct((E, 4), f32),
                   jax.ShapeDtypeStruct((E, D), f32)],
        out_specs=[_tile(TE, 4), _tile(TE, D)])
    edge2 = pl.pallas_call(
        _edge2_body, grid=(ge_steps,),
        in_specs=[_tile(TE, 4), _tile(TE, 4)],
        out_shape=jax.ShapeDtypeStruct((E, 4), f32),
        out_specs=_tile(TE, 4))
    edge3 = pl.pallas_call(
        _edge3_body, grid=(ge_steps,),
        in_specs=[_tile(TE, 4), _tile(TE, 4), _tile(TE, D), _tile(TE, D)],
        out_shape=jax.ShapeDtypeStruct((E, D), f32),
        out_specs=_tile(TE, D))
    bn1 = pl.pallas_call(
        _bn1_body, grid=(gn_steps,),
        in_specs=[_tile(TN, D), _tile(TN, D), _tile(TN, 1)],
        out_shape=[jax.ShapeDtypeStruct((N, D), f32),
                   jax.ShapeDtypeStruct((NB, D), f32)],
        out_specs=[_tile(TN, D), _full((NB, D))])
    bn2 = pl.pallas_call(
        _bn2_body, grid=(gn_steps,),
        in_specs=[_tile(TN, D), _tile(TN, 1), _full((NB, D)), _full((1, D))],
        out_shape=[jax.ShapeDtypeStruct((N, D), f32),
                   jax.ShapeDtypeStruct((NB, D), f32)],
        out_specs=[_tile(TN, D), _full((NB, D))])
    bn3 = pl.pallas_call(
        _bn3_body, grid=(gn_steps,),
        in_specs=[_tile(TN, D), _tile(TN, 1), _full((NB, D)), _full((NB, 1))]
        + [_full((1, D))] * 4,
        out_shape=jax.ShapeDtypeStruct((N, D), f32),
        out_specs=_tile(TN, D))
    readout = pl.pallas_call(
        _read_body, grid=(gn_steps,),
        in_specs=[_tile(TN, D), _tile(TN, 1)],
        out_shape=jax.ShapeDtypeStruct((NB, D), f32),
        out_specs=_full((NB, D)))

    h = x
    for l in range(L):
        q, k, v, sk = proj(h, Wq[l], bq[l].reshape(1, D), Wk[l],
                           bk[l].reshape(1, D), Wv[l], bv[l].reshape(1, D),
                           Wskip[l], bskip[l].reshape(1, D))
        alpha, ep = edge1(q[dst], k[src], ea8, We8[l], be[l].reshape(1, D))
        m = jax.ops.segment_max(alpha, dst, num_segments=N)
        m = jnp.where(jnp.isfinite(m), m, 0.0)
        ex = edge2(alpha, m[dst])
        den = jax.ops.segment_sum(ex, dst, num_segments=N)
        msg = edge3(ex, den[dst], v[src], ep)
        agg = jax.ops.segment_sum(msg, dst, num_segments=N)
        out, osum = bn1(agg, sk, batch2)
        mean = osum / ncnt
        cen, c2 = bn2(out, batch2, mean, norm_ms[l].reshape(1, D))
        var = c2 / ncnt
        h = bn3(cen, batch2, var, gf2, norm_w[l].reshape(1, D),
                norm_b[l].reshape(1, D), film_s[l].reshape(1, D),
                film_t[l].reshape(1, D))

    hG = readout(h, batch2)
    return hG @ Wout.T + bout + bias
